# baseline (speedup 1.0000x reference)
import jax
import jax.numpy as jnp
from jax import lax
from jax.experimental import pallas as pl
from jax.experimental.pallas import tpu as pltpu
from jax.experimental.pallas import tpu_sc as plsc

VOCAB = 1000000
MAXLEN = 200
D = 64
BATCH = 4096
SEQ = 200

NC, NS = 2, 16
NW = NC * NS
ROWS = BATCH * SEQ
ROWS_PER_W = ROWS // NW
CH = 160
N_CH = ROWS_PER_W // CH
NBUF = 5
N_WAVES = N_CH // NBUF
LANES = 16
KSUB = CH // LANES


def _body(x_hbm, tok_hbm, pos2_hbm, out_hbm, idx_v, rows_v, pos2_v,
          sem_g, sem_s):
    wid = lax.axis_index("s") * NC + lax.axis_index("c")
    base = wid * ROWS_PER_W

    pltpu.sync_copy(x_hbm.at[pl.ds(base, ROWS_PER_W)], idx_v)
    pltpu.sync_copy(pos2_hbm, pos2_v)

    def wave(g, carry):
        for b in range(NBUF):
            i = g * NBUF + b

            @pl.when(g > 0)
            def _wait_scatter(b=b, i=i):
                pltpu.make_async_copy(
                    rows_v.at[b], out_hbm.at[pl.ds(base + i * CH, CH)],
                    sem_s[b]
                ).wait()

            for k in range(KSUB):
                idx_vec = idx_v[pl.ds(i * CH + k * LANES, LANES)]
                pltpu.async_copy(
                    tok_hbm.at[idx_vec],
                    rows_v.at[b, pl.ds(k * LANES, LANES)], sem_g[b])

        for b in range(NBUF):
            i = g * NBUF + b
            pltpu.make_async_copy(
                tok_hbm.at[idx_v.at[pl.ds(i * CH, CH)]], rows_v.at[b],
                sem_g[b]
            ).wait()

            p0 = lax.rem(i * CH, SEQ)

            @plsc.parallel_loop(0, CH, step=1, unroll=4)
            def _add_row(r, b=b, p0=p0):
                for j in range(D // LANES):
                    sl = pl.ds(j * LANES, LANES)
                    rows_v[b, r, sl] = rows_v[b, r, sl] + pos2_v[p0 + r, sl]

            pltpu.async_copy(rows_v.at[b],
                             out_hbm.at[pl.ds(base + i * CH, CH)], sem_s[b])
        return carry

    lax.fori_loop(0, N_WAVES, wave, 0)

    for b in range(NBUF):
        pltpu.make_async_copy(
            rows_v.at[b], out_hbm.at[pl.ds(base, CH)], sem_s[b]
        ).wait()


@jax.jit
def _embed(x_flat, token_emb, pos2):
    mesh = plsc.VectorSubcoreMesh(core_axis_name="c", subcore_axis_name="s")
    f = pl.kernel(
        _body,
        out_type=jax.ShapeDtypeStruct((ROWS, D), jnp.float32),
        mesh=mesh,
        scratch_types=[
            pltpu.VMEM((ROWS_PER_W,), jnp.int32),
            pltpu.VMEM((NBUF, CH, D), jnp.float32),
            pltpu.VMEM((2 * SEQ, D), jnp.float32),
            [pltpu.SemaphoreType.DMA] * NBUF,
            [pltpu.SemaphoreType.DMA] * NBUF,
        ],
        compiler_params=pltpu.CompilerParams(use_tc_tiling_on_sc=False),
    )
    return f(x_flat, token_emb, pos2)


def kernel(x, token_emb, pos_emb):
    x_flat = x.reshape(ROWS).astype(jnp.int32)
    pos2 = jnp.concatenate([pos_emb, pos_emb], axis=0)
    out = _embed(x_flat, token_emb, pos2)
    return out.reshape(BATCH, SEQ, D)


# speedup vs baseline: 1.0411x; 1.0411x over previous
import jax
import jax.numpy as jnp
from jax import lax
from jax.experimental import pallas as pl
from jax.experimental.pallas import tpu as pltpu
from jax.experimental.pallas import tpu_sc as plsc

VOCAB = 1000000
MAXLEN = 200
D = 64
BATCH = 4096
SEQ = 200

NC, NS = 2, 16
NW = NC * NS
ROWS = BATCH * SEQ
ROWS_PER_W = ROWS // NW
CH = 128
N_CH = ROWS_PER_W // CH
NBUF = 4
N_WAVES = N_CH // NBUF
LANES = 16
KSUB = CH // LANES


def _fire_gather(idx_v, tok_hbm, gbuf, sem, i):
    for k in range(KSUB):
        idx_vec = idx_v[pl.ds(i * CH + k * LANES, LANES)]
        pltpu.async_copy(tok_hbm.at[idx_vec],
                         gbuf.at[pl.ds(k * LANES, LANES)], sem)


def _body(x_hbm, tok_hbm, pos2_hbm, out_hbm, idx_v, gbuf_v, sbuf_v, pos2_v,
          sem_g, sem_s):
    wid = lax.axis_index("s") * NC + lax.axis_index("c")
    base = wid * ROWS_PER_W

    pltpu.sync_copy(x_hbm.at[pl.ds(base, ROWS_PER_W)], idx_v)
    pltpu.sync_copy(pos2_hbm, pos2_v)

    for b in range(NBUF):
        _fire_gather(idx_v, tok_hbm, gbuf_v.at[b], sem_g[b], b)

    def wave(g, carry):
        for b in range(NBUF):
            i = g * NBUF + b
            pltpu.make_async_copy(
                tok_hbm.at[idx_v.at[pl.ds(i * CH, CH)]], gbuf_v.at[b],
                sem_g[b]
            ).wait()

            @pl.when(g > 0)
            def _wait_scatter(b=b, i=i):
                pltpu.make_async_copy(
                    sbuf_v.at[b], out_hbm.at[pl.ds(base + i * CH, CH)],
                    sem_s[b]
                ).wait()

            p0 = lax.rem(i * CH, SEQ)

            @plsc.parallel_loop(0, CH, step=1, unroll=4)
            def _add_row(r, b=b, p0=p0):
                for j in range(D // LANES):
                    sl = pl.ds(j * LANES, LANES)
                    sbuf_v[b, r, sl] = gbuf_v[b, r, sl] + pos2_v[p0 + r, sl]

            pltpu.async_copy(sbuf_v.at[b],
                             out_hbm.at[pl.ds(base + i * CH, CH)], sem_s[b])

            @pl.when(g < N_WAVES - 1)
            def _refill(b=b, i=i):
                _fire_gather(idx_v, tok_hbm, gbuf_v.at[b], sem_g[b],
                             i + NBUF)
        return carry

    lax.fori_loop(0, N_WAVES, wave, 0)

    for b in range(NBUF):
        pltpu.make_async_copy(
            sbuf_v.at[b], out_hbm.at[pl.ds(base, CH)], sem_s[b]
        ).wait()


@jax.jit
def _embed(x_flat, token_emb, pos2):
    mesh = plsc.VectorSubcoreMesh(core_axis_name="c", subcore_axis_name="s")
    f = pl.kernel(
        _body,
        out_type=jax.ShapeDtypeStruct((ROWS, D), jnp.float32),
        mesh=mesh,
        scratch_types=[
            pltpu.VMEM((ROWS_PER_W,), jnp.int32),
            pltpu.VMEM((NBUF, CH, D), jnp.float32),
            pltpu.VMEM((NBUF, CH, D), jnp.float32),
            pltpu.VMEM((2 * SEQ, D), jnp.float32),
            [pltpu.SemaphoreType.DMA] * NBUF,
            [pltpu.SemaphoreType.DMA] * NBUF,
        ],
        compiler_params=pltpu.CompilerParams(use_tc_tiling_on_sc=False),
    )
    return f(x_flat, token_emb, pos2)


def kernel(x, token_emb, pos_emb):
    x_flat = x.reshape(ROWS).astype(jnp.int32)
    pos2 = jnp.concatenate([pos_emb, pos_emb], axis=0)
    out = _embed(x_flat, token_emb, pos2)
    return out.reshape(BATCH, SEQ, D)
